# Initial kernel scaffold; baseline (speedup 1.0000x reference)
import jax
import jax.numpy as jnp
from jax.experimental import pallas as pl


def kernel(emb_2d, emb_3d, segment_ids):
    raise NotImplementedError("write your pallas kernel here")


# speedup vs baseline: 1.8229x; 1.8229x over previous
import functools

import jax
import jax.numpy as jnp
from jax import lax
from jax.experimental import pallas as pl
from jax.experimental.pallas import tpu as pltpu
from jax.experimental.pallas import tpu_sc as plsc

B = 4096
MAX_NODES = 50
D = 128
N = 102400

NC = 2
NS = 16
NW = NC * NS
NODES_PER_W = N // NW
CHUNK = 128
NCHUNK = NODES_PER_W // CHUNK
LANES = 16

ACC_ROWS = B + 128
TRASH = B
INIT_ROWS = ACC_ROWS // NS
OUT_ROWS = B // NS


def _sc_body(emb2d_hbm, emb3d_hbm, tgt_hbm,
             p2_hbm, p3_hbm,
             idx_v, buf2, buf3, acc2, acc3):
    c = lax.axis_index("c")
    s = lax.axis_index("s")
    w = s * NC + c
    node0 = w * NODES_PER_W

    def zero_row(r, carry):
        for j in range(D // LANES):
            buf2[r, pl.ds(j * LANES, LANES)] = jnp.zeros((LANES,), jnp.float32)
        return carry
    lax.fori_loop(0, CHUNK, zero_row, 0)

    ibase = s * INIT_ROWS
    for acc in (acc2, acc3):
        pltpu.sync_copy(buf2, acc.at[pl.ds(ibase, CHUNK)])
        pltpu.sync_copy(buf2, acc.at[pl.ds(ibase + CHUNK, CHUNK)])
        pltpu.sync_copy(buf2.at[pl.ds(0, INIT_ROWS - 2 * CHUNK)],
                        acc.at[pl.ds(ibase + 2 * CHUNK, INIT_ROWS - 2 * CHUNK)])

    pltpu.sync_copy(tgt_hbm.at[w], idx_v)

    plsc.subcore_barrier()

    def chunk_body(k, carry):
        nbase = node0 + k * CHUNK
        pltpu.sync_copy(emb2d_hbm.at[pl.ds(nbase, CHUNK)], buf2)
        pltpu.sync_copy(emb3d_hbm.at[pl.ds(nbase, CHUNK)], buf3)
        pltpu.sync_copy(buf2, acc2.at[idx_v.at[k]], add=True)
        pltpu.sync_copy(buf3, acc3.at[idx_v.at[k]], add=True)
        return carry
    lax.fori_loop(0, NCHUNK, chunk_body, 0)

    plsc.subcore_barrier()

    obase = s * OUT_ROWS
    pltpu.sync_copy(acc2.at[pl.ds(obase, OUT_ROWS)],
                    p2_hbm.at[c, pl.ds(obase, OUT_ROWS)])
    pltpu.sync_copy(acc3.at[pl.ds(obase, OUT_ROWS)],
                    p3_hbm.at[c, pl.ds(obase, OUT_ROWS)])


_sc_pool = pl.kernel(
    _sc_body,
    out_type=[jax.ShapeDtypeStruct((NC, B, D), jnp.float32),
              jax.ShapeDtypeStruct((NC, B, D), jnp.float32)],
    mesh=plsc.VectorSubcoreMesh(core_axis_name="c", subcore_axis_name="s"),
    scratch_types=[
        pltpu.VMEM((NCHUNK, CHUNK), jnp.int32),
        pltpu.VMEM((CHUNK, D), jnp.float32),
        pltpu.VMEM((CHUNK, D), jnp.float32),
        pltpu.VMEM_SHARED((ACC_ROWS, D), jnp.float32),
        pltpu.VMEM_SHARED((ACC_ROWS, D), jnp.float32),
    ],
)


_TC_R = 512


def _tc_body(p2_ref, p3_ref, cnt_ref, o2_ref, o3_ref):
    denom = jnp.maximum(jnp.minimum(cnt_ref[...], float(MAX_NODES)), 1.0)
    o2_ref[...] = (p2_ref[0] + p2_ref[1]) / denom
    o3_ref[...] = (p3_ref[0] + p3_ref[1]) / denom


_combine = pl.pallas_call(
    _tc_body,
    grid=(B // _TC_R,),
    in_specs=[
        pl.BlockSpec((NC, _TC_R, D), lambda i: (0, i, 0)),
        pl.BlockSpec((NC, _TC_R, D), lambda i: (0, i, 0)),
        pl.BlockSpec((_TC_R, 1), lambda i: (i, 0)),
    ],
    out_specs=[
        pl.BlockSpec((_TC_R, D), lambda i: (i, 0)),
        pl.BlockSpec((_TC_R, D), lambda i: (i, 0)),
    ],
    out_shape=[jax.ShapeDtypeStruct((B, D), jnp.float32),
               jax.ShapeDtypeStruct((B, D), jnp.float32)],
)


def kernel(emb_2d, emb_3d, segment_ids):
    seg = segment_ids.astype(jnp.int32)
    gids = jnp.arange(B, dtype=jnp.int32)
    starts = jnp.searchsorted(seg, gids, side="left").astype(jnp.int32)
    ends = jnp.concatenate([starts[1:], jnp.full((1,), N, jnp.int32)])
    cnt = (ends - starts).astype(jnp.float32).reshape(B, 1)
    pos = jnp.arange(N, dtype=jnp.int32) - starts[seg]
    tgt = jnp.where(pos < MAX_NODES, seg, TRASH).reshape(NW, NCHUNK, CHUNK)
    p2, p3 = _sc_pool(emb_2d, emb_3d, tgt)
    out2, out3 = _combine(p2, p3, cnt)
    return (out2, out3)


# speedup vs baseline: 1.8909x; 1.0373x over previous
import functools

import jax
import jax.numpy as jnp
from jax import lax
from jax.experimental import pallas as pl
from jax.experimental.pallas import tpu as pltpu
from jax.experimental.pallas import tpu_sc as plsc

B = 4096
MAX_NODES = 50
D = 128
N = 102400

NC = 2
NS = 16
NW = NC * NS
NODES_PER_W = N // NW
CHUNK = 128
NCHUNK = NODES_PER_W // CHUNK
LANES = 16

ACC_ROWS = B + NS
TRASH = B
INIT_ROWS = ACC_ROWS // NS
OUT_ROWS = B // NS


NBUF = 4


def _sc_body(emb2d_hbm, emb3d_hbm, tgt_hbm,
             p2_hbm, p3_hbm,
             idx_v, buf, acc, gsem, ssem):
    c = lax.axis_index("c")
    s = lax.axis_index("s")
    w = s * NC + c
    node0 = w * NODES_PER_W

    pltpu.sync_copy(tgt_hbm.at[w], idx_v)

    for emb_hbm, p_hbm in ((emb2d_hbm, p2_hbm), (emb3d_hbm, p3_hbm)):
        def zero_row(r, carry):
            for j in range(D // LANES):
                buf[0, r, pl.ds(j * LANES, LANES)] = (
                    jnp.zeros((LANES,), jnp.float32))
            return carry
        lax.fori_loop(0, CHUNK, zero_row, 0)

        ibase = s * INIT_ROWS
        pltpu.sync_copy(buf.at[0], acc.at[pl.ds(ibase, CHUNK)])
        pltpu.sync_copy(buf.at[0], acc.at[pl.ds(ibase + CHUNK, CHUNK)])
        pltpu.sync_copy(buf.at[0, pl.ds(0, INIT_ROWS - 2 * CHUNK)],
                        acc.at[pl.ds(ibase + 2 * CHUNK, INIT_ROWS - 2 * CHUNK)])
        plsc.subcore_barrier()

        pending_g = [None] * NBUF
        pending_s = [None] * NBUF

        def gather_start(k):
            sl = k % NBUF
            nbase = node0 + k * CHUNK
            pending_g[sl] = pltpu.async_copy(
                emb_hbm.at[pl.ds(nbase, CHUNK)], buf.at[sl], gsem[sl])

        for k in range(min(NBUF - 1, NCHUNK)):
            gather_start(k)

        for k in range(NCHUNK):
            sl = k % NBUF
            kn = k + NBUF - 1
            if kn < NCHUNK:
                sln = kn % NBUF
                if pending_s[sln] is not None:
                    pending_s[sln].wait()
                    pending_s[sln] = None
                gather_start(kn)
            pending_g[sl].wait()
            pending_g[sl] = None
            pending_s[sl] = pltpu.async_copy(
                buf.at[sl], acc.at[idx_v.at[k]], ssem[sl], add=True)

        for ps in pending_s:
            if ps is not None:
                ps.wait()

        plsc.subcore_barrier()

        obase = s * OUT_ROWS
        pltpu.sync_copy(acc.at[pl.ds(obase, OUT_ROWS)],
                        p_hbm.at[c, pl.ds(obase, OUT_ROWS)])
        plsc.subcore_barrier()


_sc_pool = pl.kernel(
    _sc_body,
    out_type=[jax.ShapeDtypeStruct((NC, B, D), jnp.float32),
              jax.ShapeDtypeStruct((NC, B, D), jnp.float32)],
    mesh=plsc.VectorSubcoreMesh(core_axis_name="c", subcore_axis_name="s"),
    scratch_types=[
        pltpu.VMEM((NCHUNK, CHUNK), jnp.int32),
        pltpu.VMEM((NBUF, CHUNK, D), jnp.float32),
        pltpu.VMEM_SHARED((ACC_ROWS, D), jnp.float32),
        [pltpu.SemaphoreType.DMA] * NBUF,
        [pltpu.SemaphoreType.DMA] * NBUF,
    ],
)


_TC_R = 512


def _tc_body(p2_ref, p3_ref, cnt_ref, o2_ref, o3_ref):
    denom = jnp.maximum(jnp.minimum(cnt_ref[...], float(MAX_NODES)), 1.0)
    o2_ref[...] = (p2_ref[0] + p2_ref[1]) / denom
    o3_ref[...] = (p3_ref[0] + p3_ref[1]) / denom


_combine = pl.pallas_call(
    _tc_body,
    grid=(B // _TC_R,),
    in_specs=[
        pl.BlockSpec((NC, _TC_R, D), lambda i: (0, i, 0)),
        pl.BlockSpec((NC, _TC_R, D), lambda i: (0, i, 0)),
        pl.BlockSpec((_TC_R, 1), lambda i: (i, 0)),
    ],
    out_specs=[
        pl.BlockSpec((_TC_R, D), lambda i: (i, 0)),
        pl.BlockSpec((_TC_R, D), lambda i: (i, 0)),
    ],
    out_shape=[jax.ShapeDtypeStruct((B, D), jnp.float32),
               jax.ShapeDtypeStruct((B, D), jnp.float32)],
)


def kernel(emb_2d, emb_3d, segment_ids):
    seg = segment_ids.astype(jnp.int32)
    gids = jnp.arange(B, dtype=jnp.int32)
    starts = jnp.searchsorted(seg, gids, side="left").astype(jnp.int32)
    ends = jnp.concatenate([starts[1:], jnp.full((1,), N, jnp.int32)])
    cnt = (ends - starts).astype(jnp.float32).reshape(B, 1)
    pos = jnp.arange(N, dtype=jnp.int32) - starts[seg]
    tgt = jnp.where(pos < MAX_NODES, seg, TRASH).reshape(NW, NCHUNK, CHUNK)
    p2, p3 = _sc_pool(emb_2d, emb_3d, tgt)
    out2, out3 = _combine(p2, p3, cnt)
    return (out2, out3)


# speedup vs baseline: 9.0327x; 4.7768x over previous
import functools

import jax
import jax.numpy as jnp
from jax import lax
from jax.experimental import pallas as pl
from jax.experimental.pallas import tpu as pltpu
from jax.experimental.pallas import tpu_sc as plsc

B = 4096
MAX_NODES = 50
D = 128
N = 102400

NC = 2
NS = 16
NW = NC * NS
NODES_PER_W = N // NW
CHUNK = 128
NCHUNK = NODES_PER_W // CHUNK
LANES = 16

ACC_ROWS = B + NS
TRASH = B
INIT_ROWS = ACC_ROWS // NS
OUT_ROWS = B // NS


NBUF = 3


def _sc_body(emb2d_hbm, emb3d_hbm, tgt_hbm,
             p2_hbm, p3_hbm,
             idx_v, buf, acc, gsem, ssem):
    c = lax.axis_index("c")
    s = lax.axis_index("s")
    w = s * NC + c
    node0 = w * NODES_PER_W

    pltpu.sync_copy(tgt_hbm.at[w], idx_v)

    ibase = s * INIT_ROWS
    obase = s * OUT_ROWS

    for emb_hbm, p_hbm in ((emb2d_hbm, p2_hbm), (emb3d_hbm, p3_hbm)):
        def zero_row(r, carry):
            for j in range(D // LANES):
                buf[0, r, pl.ds(j * LANES, LANES)] = (
                    jnp.zeros((LANES,), jnp.float32))
            return carry
        lax.fori_loop(0, CHUNK, zero_row, 0)

        pltpu.sync_copy(buf.at[0], acc.at[pl.ds(ibase, CHUNK)])
        pltpu.sync_copy(buf.at[0], acc.at[pl.ds(ibase + CHUNK, CHUNK)])
        pltpu.sync_copy(buf.at[0, pl.ds(0, INIT_ROWS - 2 * CHUNK)],
                        acc.at[pl.ds(ibase + 2 * CHUNK, INIT_ROWS - 2 * CHUNK)])
        plsc.subcore_barrier()

        pending_g = [None] * NBUF
        pending_s = [None] * NBUF

        def gather_start(k):
            sl = k % NBUF
            nbase = node0 + k * CHUNK
            pending_g[sl] = pltpu.async_copy(
                emb_hbm.at[pl.ds(nbase, CHUNK)], buf.at[sl], gsem[sl])

        for k in range(min(NBUF - 1, NCHUNK)):
            gather_start(k)

        for k in range(NCHUNK):
            sl = k % NBUF
            kn = k + NBUF - 1
            if kn < NCHUNK:
                sln = kn % NBUF
                if pending_s[sln] is not None:
                    pending_s[sln].wait()
                    pending_s[sln] = None
                gather_start(kn)
            pending_g[sl].wait()
            pending_g[sl] = None
            pending_s[sl] = pltpu.async_copy(
                buf.at[sl], acc.at[idx_v.at[k]], ssem[sl], add=True)

        for ps in pending_s:
            if ps is not None:
                ps.wait()

        plsc.subcore_barrier()

        pltpu.sync_copy(acc.at[pl.ds(obase, OUT_ROWS)],
                        p_hbm.at[c, pl.ds(obase, OUT_ROWS)])
        plsc.subcore_barrier()


_sc_pool = pl.kernel(
    _sc_body,
    out_type=[jax.ShapeDtypeStruct((NC, B, D), jnp.float32),
              jax.ShapeDtypeStruct((NC, B, D), jnp.float32)],
    mesh=plsc.VectorSubcoreMesh(core_axis_name="c", subcore_axis_name="s"),
    scratch_types=[
        pltpu.VMEM((NCHUNK, CHUNK), jnp.int32),
        pltpu.VMEM((NBUF, CHUNK, D), jnp.float32),
        pltpu.VMEM_SHARED((ACC_ROWS, D), jnp.float32),
        [pltpu.SemaphoreType.DMA] * NBUF,
        [pltpu.SemaphoreType.DMA] * NBUF,
    ],
)


_TC_R = 512


def _tc_body(p2_ref, p3_ref, cnt_ref, o2_ref, o3_ref):
    denom = jnp.maximum(cnt_ref[...], 1.0)
    o2_ref[...] = (p2_ref[0] + p2_ref[1]) / denom
    o3_ref[...] = (p3_ref[0] + p3_ref[1]) / denom


_combine = pl.pallas_call(
    _tc_body,
    grid=(B // _TC_R,),
    in_specs=[
        pl.BlockSpec((NC, _TC_R, D), lambda i: (0, i, 0)),
        pl.BlockSpec((NC, _TC_R, D), lambda i: (0, i, 0)),
        pl.BlockSpec((_TC_R, 1), lambda i: (i, 0)),
    ],
    out_specs=[
        pl.BlockSpec((_TC_R, D), lambda i: (i, 0)),
        pl.BlockSpec((_TC_R, D), lambda i: (i, 0)),
    ],
    out_shape=[jax.ShapeDtypeStruct((B, D), jnp.float32),
               jax.ShapeDtypeStruct((B, D), jnp.float32)],
)


def kernel(emb_2d, emb_3d, segment_ids):
    seg = segment_ids.astype(jnp.int32)
    i = jnp.arange(N, dtype=jnp.int32)
    boundary = jnp.concatenate(
        [jnp.ones((1,), jnp.bool_), seg[1:] != seg[:-1]])
    run_start = lax.cummax(jnp.where(boundary, i, 0))
    pos = i - run_start
    valid = pos < MAX_NODES
    tgt = jnp.where(valid, seg, TRASH).reshape(NW, NCHUNK, CHUNK)
    cnt = jnp.zeros((B,), jnp.float32).at[seg].add(
        valid.astype(jnp.float32), mode="drop").reshape(B, 1)
    p2, p3 = _sc_pool(emb_2d, emb_3d, tgt)
    out2, out3 = _combine(p2, p3, cnt)
    return (out2, out3)


# speedup vs baseline: 15.4989x; 1.7159x over previous
import functools

import jax
import jax.numpy as jnp
from jax import lax
from jax.experimental import pallas as pl
from jax.experimental.pallas import tpu as pltpu
from jax.experimental.pallas import tpu_sc as plsc

B = 4096
MAX_NODES = 50
D = 128
N = 102400

NC = 2
NS = 16
NW = NC * NS
NODES_PER_W = N // NW
CHUNK = 128
NCHUNK = NODES_PER_W // CHUNK
LANES = 16

ACC_ROWS = B + NS
TRASH = B
INIT_ROWS = ACC_ROWS // NS
OUT_ROWS = B // NS


NBUF = 2
CNT_W = 128


def _sc_body(emb2d_hbm, emb3d_hbm, tgt_hbm,
             p2_hbm, p3_hbm, cnt_hbm,
             idx_v, buf, ones_v, acc, cacc, gsem, ssem, csem):
    c = lax.axis_index("c")
    s = lax.axis_index("s")
    w = s * NC + c
    node0 = w * NODES_PER_W

    pltpu.sync_copy(tgt_hbm.at[w], idx_v)

    ibase = s * INIT_ROWS
    obase = s * OUT_ROWS

    def fill_ones(r, carry):
        for j in range(CNT_W // LANES):
            ones_v[r, pl.ds(j * LANES, LANES)] = jnp.ones((LANES,), jnp.float32)
        return carry
    lax.fori_loop(0, CHUNK, fill_ones, 0)

    for passno, (emb_hbm, p_hbm) in enumerate(
            ((emb2d_hbm, p2_hbm), (emb3d_hbm, p3_hbm))):
        def zero_row(r, carry):
            for j in range(D // LANES):
                buf[0, r, pl.ds(j * LANES, LANES)] = (
                    jnp.zeros((LANES,), jnp.float32))
            return carry
        lax.fori_loop(0, CHUNK, zero_row, 0)

        pltpu.sync_copy(buf.at[0], acc.at[pl.ds(ibase, CHUNK)])
        pltpu.sync_copy(buf.at[0], acc.at[pl.ds(ibase + CHUNK, CHUNK)])
        pltpu.sync_copy(buf.at[0, pl.ds(0, INIT_ROWS - 2 * CHUNK)],
                        acc.at[pl.ds(ibase + 2 * CHUNK, INIT_ROWS - 2 * CHUNK)])
        if passno == 0:
            pltpu.sync_copy(buf.at[0], cacc.at[pl.ds(ibase, CHUNK)])
            pltpu.sync_copy(buf.at[0], cacc.at[pl.ds(ibase + CHUNK, CHUNK)])
            pltpu.sync_copy(buf.at[0, pl.ds(0, INIT_ROWS - 2 * CHUNK)],
                            cacc.at[pl.ds(ibase + 2 * CHUNK,
                                          INIT_ROWS - 2 * CHUNK)])
        plsc.subcore_barrier()

        pending_g = [None] * NBUF
        pending_s = [None] * NBUF
        pending_c = [None] * NBUF

        def gather_start(k):
            sl = k % NBUF
            nbase = node0 + k * CHUNK
            pending_g[sl] = pltpu.async_copy(
                emb_hbm.at[pl.ds(nbase, CHUNK)], buf.at[sl], gsem[sl])

        for k in range(min(NBUF - 1, NCHUNK)):
            gather_start(k)

        for k in range(NCHUNK):
            sl = k % NBUF
            kn = k + NBUF - 1
            if kn < NCHUNK:
                sln = kn % NBUF
                if pending_s[sln] is not None:
                    pending_s[sln].wait()
                    pending_s[sln] = None
                gather_start(kn)
            pending_g[sl].wait()
            pending_g[sl] = None
            pending_s[sl] = pltpu.async_copy(
                buf.at[sl], acc.at[idx_v.at[k]], ssem[sl], add=True)
            if passno == 0:
                if pending_c[sl] is not None:
                    pending_c[sl].wait()
                pending_c[sl] = pltpu.async_copy(
                    ones_v, cacc.at[idx_v.at[k]], csem[sl], add=True)

        for ps in pending_s:
            if ps is not None:
                ps.wait()
        for pc in pending_c:
            if pc is not None:
                pc.wait()

        plsc.subcore_barrier()

        pltpu.sync_copy(acc.at[pl.ds(obase, OUT_ROWS)],
                        p_hbm.at[c, pl.ds(obase, OUT_ROWS)])
        if passno == 0:
            pltpu.sync_copy(cacc.at[pl.ds(obase, OUT_ROWS)],
                            cnt_hbm.at[c, pl.ds(obase, OUT_ROWS)])
        plsc.subcore_barrier()


_sc_pool = pl.kernel(
    _sc_body,
    out_type=[jax.ShapeDtypeStruct((NC, B, D), jnp.float32),
              jax.ShapeDtypeStruct((NC, B, D), jnp.float32),
              jax.ShapeDtypeStruct((NC, B, CNT_W), jnp.float32)],
    mesh=plsc.VectorSubcoreMesh(core_axis_name="c", subcore_axis_name="s"),
    scratch_types=[
        pltpu.VMEM((NCHUNK, CHUNK), jnp.int32),
        pltpu.VMEM((NBUF, CHUNK, D), jnp.float32),
        pltpu.VMEM((CHUNK, CNT_W), jnp.float32),
        pltpu.VMEM_SHARED((ACC_ROWS, D), jnp.float32),
        pltpu.VMEM_SHARED((ACC_ROWS, CNT_W), jnp.float32),
        [pltpu.SemaphoreType.DMA] * NBUF,
        [pltpu.SemaphoreType.DMA] * NBUF,
        [pltpu.SemaphoreType.DMA] * NBUF,
    ],
)


_TC_R = 512


def _tc_body(p2_ref, p3_ref, cnt_ref, o2_ref, o3_ref):
    denom = jnp.maximum(cnt_ref[0, :, :1] + cnt_ref[1, :, :1], 1.0)
    o2_ref[...] = (p2_ref[0] + p2_ref[1]) / denom
    o3_ref[...] = (p3_ref[0] + p3_ref[1]) / denom


_combine = pl.pallas_call(
    _tc_body,
    grid=(B // _TC_R,),
    in_specs=[
        pl.BlockSpec((NC, _TC_R, D), lambda i: (0, i, 0)),
        pl.BlockSpec((NC, _TC_R, D), lambda i: (0, i, 0)),
        pl.BlockSpec((NC, _TC_R, CNT_W), lambda i: (0, i, 0)),
    ],
    out_specs=[
        pl.BlockSpec((_TC_R, D), lambda i: (i, 0)),
        pl.BlockSpec((_TC_R, D), lambda i: (i, 0)),
    ],
    out_shape=[jax.ShapeDtypeStruct((B, D), jnp.float32),
               jax.ShapeDtypeStruct((B, D), jnp.float32)],
)


def kernel(emb_2d, emb_3d, segment_ids):
    seg = segment_ids.astype(jnp.int32)
    i = jnp.arange(N, dtype=jnp.int32)
    boundary = jnp.concatenate(
        [jnp.ones((1,), jnp.bool_), seg[1:] != seg[:-1]])
    run_start = lax.cummax(jnp.where(boundary, i, 0))
    pos = i - run_start
    tgt = jnp.where(pos < MAX_NODES, seg, TRASH).reshape(NW, NCHUNK, CHUNK)
    p2, p3, cnt = _sc_pool(emb_2d, emb_3d, tgt)
    out2, out3 = _combine(p2, p3, cnt)
    return (out2, out3)


# speedup vs baseline: 15.6388x; 1.0090x over previous
import functools

import jax
import jax.numpy as jnp
from jax import lax
from jax.experimental import pallas as pl
from jax.experimental.pallas import tpu as pltpu
from jax.experimental.pallas import tpu_sc as plsc

B = 4096
MAX_NODES = 50
D = 128
N = 102400

NC = 2
NS = 16
NW = NC * NS
NODES_PER_W = N // NW
CHUNK = 128
NCHUNK = NODES_PER_W // CHUNK
LANES = 16

ACC_ROWS = B + NS
TRASH = B
INIT_ROWS = ACC_ROWS // NS
OUT_ROWS = B // NS


NBUF = 2
CNT_W = 128


def _sc_body(emb2d_hbm, emb3d_hbm, tgt_hbm,
             p2_hbm, p3_hbm, cnt_hbm,
             idx_v, buf, ones_v, acc, cacc, gsem, ssem, csem):
    c = lax.axis_index("c")
    s = lax.axis_index("s")
    w = s * NC + c
    node0 = w * NODES_PER_W

    pltpu.sync_copy(tgt_hbm.at[w], idx_v)

    ibase = s * INIT_ROWS
    obase = s * OUT_ROWS

    def fill_ones(r, carry):
        for j in range(CNT_W // LANES):
            ones_v[r, pl.ds(j * LANES, LANES)] = jnp.ones((LANES,), jnp.float32)
        return carry
    lax.fori_loop(0, CHUNK, fill_ones, 0)

    CSPLIT = NCHUNK // 2
    for passno, (emb_hbm, p_hbm) in enumerate(
            ((emb2d_hbm, p2_hbm), (emb3d_hbm, p3_hbm))):
        if passno == 0:
            def zero_row(r, carry):
                for j in range(D // LANES):
                    buf[0, r, pl.ds(j * LANES, LANES)] = (
                        jnp.zeros((LANES,), jnp.float32))
                return carry
            lax.fori_loop(0, CHUNK, zero_row, 0)

            for a in (acc, cacc):
                pltpu.sync_copy(buf.at[0], a.at[pl.ds(ibase, CHUNK)])
                pltpu.sync_copy(buf.at[0], a.at[pl.ds(ibase + CHUNK, CHUNK)])
                pltpu.sync_copy(buf.at[0, pl.ds(0, INIT_ROWS - 2 * CHUNK)],
                                a.at[pl.ds(ibase + 2 * CHUNK,
                                           INIT_ROWS - 2 * CHUNK)])
            plsc.subcore_barrier()

        pending_g = [None] * NBUF
        pending_s = [None] * NBUF
        pending_c = [None] * NBUF

        def gather_start(k):
            sl = k % NBUF
            nbase = node0 + k * CHUNK
            pending_g[sl] = pltpu.async_copy(
                emb_hbm.at[pl.ds(nbase, CHUNK)], buf.at[sl], gsem[sl])

        for k in range(min(NBUF - 1, NCHUNK)):
            gather_start(k)

        for k in range(NCHUNK):
            sl = k % NBUF
            kn = k + NBUF - 1
            if kn < NCHUNK:
                sln = kn % NBUF
                if pending_s[sln] is not None:
                    pending_s[sln].wait()
                    pending_s[sln] = None
                gather_start(kn)
            pending_g[sl].wait()
            pending_g[sl] = None
            pending_s[sl] = pltpu.async_copy(
                buf.at[sl], acc.at[idx_v.at[k]], ssem[sl], add=True)
            if (passno == 0) == (k < CSPLIT):
                if pending_c[sl] is not None:
                    pending_c[sl].wait()
                pending_c[sl] = pltpu.async_copy(
                    ones_v, cacc.at[idx_v.at[k]], csem[sl], add=True)

        for ps in pending_s:
            if ps is not None:
                ps.wait()
        for pc in pending_c:
            if pc is not None:
                pc.wait()

        plsc.subcore_barrier()

        pltpu.sync_copy(acc.at[pl.ds(obase, OUT_ROWS)],
                        p_hbm.at[c, pl.ds(obase, OUT_ROWS)])
        if passno == 0:
            plsc.subcore_barrier()
        else:
            pltpu.sync_copy(cacc.at[pl.ds(obase, OUT_ROWS)],
                            cnt_hbm.at[c, pl.ds(obase, OUT_ROWS)])


_sc_pool = pl.kernel(
    _sc_body,
    out_type=[jax.ShapeDtypeStruct((NC, B, D), jnp.float32),
              jax.ShapeDtypeStruct((NC, B, D), jnp.float32),
              jax.ShapeDtypeStruct((NC, B, CNT_W), jnp.float32)],
    mesh=plsc.VectorSubcoreMesh(core_axis_name="c", subcore_axis_name="s"),
    scratch_types=[
        pltpu.VMEM((NCHUNK, CHUNK), jnp.int32),
        pltpu.VMEM((NBUF, CHUNK, D), jnp.float32),
        pltpu.VMEM((CHUNK, CNT_W), jnp.float32),
        pltpu.VMEM_SHARED((ACC_ROWS, D), jnp.float32),
        pltpu.VMEM_SHARED((ACC_ROWS, CNT_W), jnp.float32),
        [pltpu.SemaphoreType.DMA] * NBUF,
        [pltpu.SemaphoreType.DMA] * NBUF,
        [pltpu.SemaphoreType.DMA] * NBUF,
    ],
)


_TC_R = 512


def _tc_body(p2_ref, p23_ref, cnt_ref, o2_ref, o3_ref):
    denom = jnp.maximum(cnt_ref[0, :, :1] + cnt_ref[1, :, :1], 1.0)
    s2 = p2_ref[0] + p2_ref[1]
    o2_ref[...] = s2 / denom
    o3_ref[...] = (p23_ref[0] + p23_ref[1] - s2) / denom


_combine = pl.pallas_call(
    _tc_body,
    grid=(B // _TC_R,),
    in_specs=[
        pl.BlockSpec((NC, _TC_R, D), lambda i: (0, i, 0)),
        pl.BlockSpec((NC, _TC_R, D), lambda i: (0, i, 0)),
        pl.BlockSpec((NC, _TC_R, CNT_W), lambda i: (0, i, 0)),
    ],
    out_specs=[
        pl.BlockSpec((_TC_R, D), lambda i: (i, 0)),
        pl.BlockSpec((_TC_R, D), lambda i: (i, 0)),
    ],
    out_shape=[jax.ShapeDtypeStruct((B, D), jnp.float32),
               jax.ShapeDtypeStruct((B, D), jnp.float32)],
)


def kernel(emb_2d, emb_3d, segment_ids):
    seg = segment_ids.astype(jnp.int32)
    i = jnp.arange(N, dtype=jnp.int32)
    boundary = jnp.concatenate(
        [jnp.ones((1,), jnp.bool_), seg[1:] != seg[:-1]])
    run_start = lax.cummax(jnp.where(boundary, i, 0))
    pos = i - run_start
    tgt = jnp.where(pos < MAX_NODES, seg, TRASH).reshape(NW, NCHUNK, CHUNK)
    p2, p3, cnt = _sc_pool(emb_2d, emb_3d, tgt)
    out2, out3 = _combine(p2, p3, cnt)
    return (out2, out3)


# speedup vs baseline: 15.6600x; 1.0014x over previous
import functools

import jax
import jax.numpy as jnp
from jax import lax
from jax.experimental import pallas as pl
from jax.experimental.pallas import tpu as pltpu
from jax.experimental.pallas import tpu_sc as plsc

B = 4096
MAX_NODES = 50
D = 128
N = 102400

NC = 2
NS = 16
NW = NC * NS
NODES_PER_W = N // NW
CHUNK = 128
NCHUNK = NODES_PER_W // CHUNK
LANES = 16

ACC_ROWS = B + NS
TRASH = B
INIT_ROWS = ACC_ROWS // NS
OUT_ROWS = B // NS


NBUF = 2
CNT_W = 128


def _sc_body(emb2d_hbm, emb3d_hbm, tgt_hbm,
             p2_hbm, p3_hbm, cnt_hbm,
             idx_v, buf, ones_v, acc, cacc, gsem, ssem, csem):
    c = lax.axis_index("c")
    s = lax.axis_index("s")
    w = s * NC + c
    node0 = w * NODES_PER_W

    pltpu.sync_copy(tgt_hbm.at[w], idx_v)

    ibase = s * INIT_ROWS
    obase = s * OUT_ROWS

    def fill_ones_v(val):
        def body(r, carry):
            for j in range(CNT_W // LANES):
                ones_v[r, pl.ds(j * LANES, LANES)] = jnp.full(
                    (LANES,), val, jnp.float32)
            return carry
        lax.fori_loop(0, CHUNK, body, 0)

    CSPLIT = NCHUNK // 2
    for passno, (emb_hbm, p_hbm) in enumerate(
            ((emb2d_hbm, p2_hbm), (emb3d_hbm, p3_hbm))):
        if passno == 0:
            def zero_row(r, carry):
                for j in range(D // LANES):
                    buf[0, r, pl.ds(j * LANES, LANES)] = (
                        jnp.zeros((LANES,), jnp.float32))
                return carry
            lax.fori_loop(0, CHUNK, zero_row, 0)

            pltpu.sync_copy(buf.at[0], acc.at[pl.ds(ibase, CHUNK)])
            pltpu.sync_copy(buf.at[0], acc.at[pl.ds(ibase + CHUNK, CHUNK)])
            pltpu.sync_copy(buf.at[0, pl.ds(0, INIT_ROWS - 2 * CHUNK)],
                            acc.at[pl.ds(ibase + 2 * CHUNK,
                                         INIT_ROWS - 2 * CHUNK)])
            pltpu.sync_copy(buf.at[0], cacc.at[pl.ds(ibase, CHUNK)])
            pltpu.sync_copy(buf.at[0], cacc.at[pl.ds(ibase + CHUNK, CHUNK)])
            pltpu.sync_copy(buf.at[0, pl.ds(0, INIT_ROWS - 2 * CHUNK)],
                            cacc.at[pl.ds(ibase + 2 * CHUNK,
                                          INIT_ROWS - 2 * CHUNK)])
            fill_ones_v(1.0)
            plsc.subcore_barrier()

        pending_g = [None] * NBUF
        pending_s = [None] * NBUF
        pending_c = [None] * NBUF

        def gather_start(k):
            sl = k % NBUF
            nbase = node0 + k * CHUNK
            pending_g[sl] = pltpu.async_copy(
                emb_hbm.at[pl.ds(nbase, CHUNK)], buf.at[sl], gsem[sl])

        for k in range(min(NBUF - 1, NCHUNK)):
            gather_start(k)

        for k in range(NCHUNK):
            sl = k % NBUF
            kn = k + NBUF - 1
            if kn < NCHUNK:
                sln = kn % NBUF
                if pending_s[sln] is not None:
                    pending_s[sln].wait()
                    pending_s[sln] = None
                gather_start(kn)
            pending_g[sl].wait()
            pending_g[sl] = None
            pending_s[sl] = pltpu.async_copy(
                buf.at[sl], acc.at[idx_v.at[k]], ssem[sl], add=True)
            if (passno == 0) == (k < CSPLIT):
                if pending_c[sl] is not None:
                    pending_c[sl].wait()
                pending_c[sl] = pltpu.async_copy(
                    ones_v, cacc.at[idx_v.at[k]], csem[sl], add=True)

        for ps in pending_s:
            if ps is not None:
                ps.wait()
        for pc in pending_c:
            if pc is not None:
                pc.wait()

        plsc.subcore_barrier()

        pltpu.sync_copy(acc.at[pl.ds(obase, OUT_ROWS)],
                        p_hbm.at[c, pl.ds(obase, OUT_ROWS)])
        if passno == 0:
            plsc.subcore_barrier()
        else:
            pltpu.sync_copy(cacc.at[pl.ds(obase, OUT_ROWS)],
                            cnt_hbm.at[c, pl.ds(obase, OUT_ROWS)])


_sc_pool = pl.kernel(
    _sc_body,
    out_type=[jax.ShapeDtypeStruct((NC, B, D), jnp.float32),
              jax.ShapeDtypeStruct((NC, B, D), jnp.float32),
              jax.ShapeDtypeStruct((NC, B, CNT_W), jnp.float32)],
    mesh=plsc.VectorSubcoreMesh(core_axis_name="c", subcore_axis_name="s"),
    scratch_types=[
        pltpu.VMEM((NCHUNK, CHUNK), jnp.int32),
        pltpu.VMEM((NBUF, CHUNK, D), jnp.float32),
        pltpu.VMEM((CHUNK, CNT_W), jnp.float32),
        pltpu.VMEM_SHARED((ACC_ROWS, D), jnp.float32),
        pltpu.VMEM_SHARED((ACC_ROWS, CNT_W), jnp.float32),
        [pltpu.SemaphoreType.DMA] * NBUF,
        [pltpu.SemaphoreType.DMA] * NBUF,
        [pltpu.SemaphoreType.DMA] * NBUF,
    ],
)


_TC_R = 512


def _tc_body(p2_ref, p23_ref, cnt_ref, o2_ref, o3_ref):
    denom = jnp.maximum(cnt_ref[0, :, :1] + cnt_ref[1, :, :1], 1.0)
    s2 = p2_ref[0] + p2_ref[1]
    o2_ref[...] = s2 / denom
    o3_ref[...] = (p23_ref[0] + p23_ref[1] - s2) / denom


_combine = pl.pallas_call(
    _tc_body,
    grid=(B // _TC_R,),
    in_specs=[
        pl.BlockSpec((NC, _TC_R, D), lambda i: (0, i, 0)),
        pl.BlockSpec((NC, _TC_R, D), lambda i: (0, i, 0)),
        pl.BlockSpec((NC, _TC_R, CNT_W), lambda i: (0, i, 0)),
    ],
    out_specs=[
        pl.BlockSpec((_TC_R, D), lambda i: (i, 0)),
        pl.BlockSpec((_TC_R, D), lambda i: (i, 0)),
    ],
    out_shape=[jax.ShapeDtypeStruct((B, D), jnp.float32),
               jax.ShapeDtypeStruct((B, D), jnp.float32)],
)


def kernel(emb_2d, emb_3d, segment_ids):
    seg = segment_ids.astype(jnp.int32)
    i = jnp.arange(N, dtype=jnp.int32)
    boundary = jnp.concatenate(
        [jnp.ones((1,), jnp.bool_), seg[1:] != seg[:-1]])
    run_start = lax.cummax(jnp.where(boundary, i, 0))
    pos = i - run_start
    tgt = jnp.where(pos < MAX_NODES, seg, TRASH).reshape(NW, NCHUNK, CHUNK)
    p2, p3, cnt = _sc_pool(emb_2d, emb_3d, tgt)
    out2, out3 = _combine(p2, p3, cnt)
    return (out2, out3)


# speedup vs baseline: 15.6944x; 1.0022x over previous
import jax
import jax.numpy as jnp
from jax import lax
from jax.experimental import pallas as pl
from jax.experimental.pallas import tpu as pltpu
from jax.experimental.pallas import tpu_sc as plsc

B = 4096
MAX_NODES = 50
D = 128
N = 102400

NC = 2
NS = 16
NW = NC * NS
NODES_PER_W = N // NW
CHUNK = 128
NCHUNK = NODES_PER_W // CHUNK
LANES = 16

ACC_ROWS = B + NS
TRASH = B
INIT_ROWS = ACC_ROWS // NS
OUT_ROWS = B // NS


NBUF = 2
CNT_W = 128


def _sc_body(emb2d_hbm, emb3d_hbm, tgt_hbm,
             p2_hbm, p3_hbm, cnt_hbm,
             idx_v, buf, ones_v, acc, cacc, gsem, ssem, csem):
    c = lax.axis_index("c")
    s = lax.axis_index("s")
    w = s * NC + c
    node0 = w * NODES_PER_W

    pltpu.sync_copy(tgt_hbm.at[w], idx_v)

    ibase = s * INIT_ROWS
    obase = s * OUT_ROWS

    def fill_ones_v(val):
        def body(r, carry):
            for j in range(CNT_W // LANES):
                ones_v[r, pl.ds(j * LANES, LANES)] = jnp.full(
                    (LANES,), val, jnp.float32)
            return carry
        lax.fori_loop(0, CHUNK, body, 0)

    CSPLIT = NCHUNK // 2
    for passno, (emb_hbm, p_hbm) in enumerate(
            ((emb2d_hbm, p2_hbm), (emb3d_hbm, p3_hbm))):
        if passno == 0:
            def zero_row(r, carry):
                for j in range(D // LANES):
                    buf[0, r, pl.ds(j * LANES, LANES)] = (
                        jnp.zeros((LANES,), jnp.float32))
                return carry
            lax.fori_loop(0, CHUNK, zero_row, 0)

            pltpu.sync_copy(buf.at[0], acc.at[pl.ds(ibase, CHUNK)])
            pltpu.sync_copy(buf.at[0], acc.at[pl.ds(ibase + CHUNK, CHUNK)])
            pltpu.sync_copy(buf.at[0, pl.ds(0, INIT_ROWS - 2 * CHUNK)],
                            acc.at[pl.ds(ibase + 2 * CHUNK,
                                         INIT_ROWS - 2 * CHUNK)])
            pltpu.sync_copy(buf.at[0], cacc.at[pl.ds(ibase, CHUNK)])
            pltpu.sync_copy(buf.at[0], cacc.at[pl.ds(ibase + CHUNK, CHUNK)])
            pltpu.sync_copy(buf.at[0, pl.ds(0, INIT_ROWS - 2 * CHUNK)],
                            cacc.at[pl.ds(ibase + 2 * CHUNK,
                                          INIT_ROWS - 2 * CHUNK)])
            fill_ones_v(1.0)
            plsc.subcore_barrier()

        pending_g = [None] * NBUF
        pending_s = [None] * NBUF
        pending_c = [None] * NBUF

        def gather_start(k):
            sl = k % NBUF
            nbase = node0 + k * CHUNK
            pending_g[sl] = pltpu.async_copy(
                emb_hbm.at[pl.ds(nbase, CHUNK)], buf.at[sl], gsem[sl])

        for k in range(min(NBUF - 1, NCHUNK)):
            gather_start(k)

        for k in range(NCHUNK):
            sl = k % NBUF
            kn = k + NBUF - 1
            if kn < NCHUNK:
                sln = kn % NBUF
                if pending_s[sln] is not None:
                    pending_s[sln].wait()
                    pending_s[sln] = None
                gather_start(kn)
            pending_g[sl].wait()
            pending_g[sl] = None
            pending_s[sl] = pltpu.async_copy(
                buf.at[sl], acc.at[idx_v.at[k]], ssem[sl], add=True)
            if (passno == 0) == (k < CSPLIT):
                if pending_c[sl] is not None:
                    pending_c[sl].wait()
                pending_c[sl] = pltpu.async_copy(
                    ones_v, cacc.at[idx_v.at[k]], csem[sl], add=True)

        for ps in pending_s:
            if ps is not None:
                ps.wait()
        for pc in pending_c:
            if pc is not None:
                pc.wait()

        plsc.subcore_barrier()

        pltpu.sync_copy(acc.at[pl.ds(obase, OUT_ROWS)],
                        p_hbm.at[c, pl.ds(obase, OUT_ROWS)])
        if passno == 0:
            plsc.subcore_barrier()
        else:
            pltpu.sync_copy(cacc.at[pl.ds(obase, OUT_ROWS)],
                            cnt_hbm.at[c, pl.ds(obase, OUT_ROWS)])


_sc_pool = pl.kernel(
    _sc_body,
    out_type=[jax.ShapeDtypeStruct((NC, B, D), jnp.float32),
              jax.ShapeDtypeStruct((NC, B, D), jnp.float32),
              jax.ShapeDtypeStruct((NC, B, CNT_W), jnp.float32)],
    mesh=plsc.VectorSubcoreMesh(core_axis_name="c", subcore_axis_name="s"),
    scratch_types=[
        pltpu.VMEM((NCHUNK, CHUNK), jnp.int32),
        pltpu.VMEM((NBUF, CHUNK, D), jnp.float32),
        pltpu.VMEM((CHUNK, CNT_W), jnp.float32),
        pltpu.VMEM_SHARED((ACC_ROWS, D), jnp.float32),
        pltpu.VMEM_SHARED((ACC_ROWS, CNT_W), jnp.float32),
        [pltpu.SemaphoreType.DMA] * NBUF,
        [pltpu.SemaphoreType.DMA] * NBUF,
        [pltpu.SemaphoreType.DMA] * NBUF,
    ],
)


_TC_R = 512


def _tc_body(p2_ref, p23_ref, cnt_ref, o2_ref, o3_ref):
    denom = jnp.maximum(cnt_ref[0, :, :1] + cnt_ref[1, :, :1], 1.0)
    s2 = p2_ref[0] + p2_ref[1]
    o2_ref[...] = s2 / denom
    o3_ref[...] = (p23_ref[0] + p23_ref[1] - s2) / denom


_combine = pl.pallas_call(
    _tc_body,
    grid=(B // _TC_R,),
    in_specs=[
        pl.BlockSpec((NC, _TC_R, D), lambda i: (0, i, 0)),
        pl.BlockSpec((NC, _TC_R, D), lambda i: (0, i, 0)),
        pl.BlockSpec((NC, _TC_R, CNT_W), lambda i: (0, i, 0)),
    ],
    out_specs=[
        pl.BlockSpec((_TC_R, D), lambda i: (i, 0)),
        pl.BlockSpec((_TC_R, D), lambda i: (i, 0)),
    ],
    out_shape=[jax.ShapeDtypeStruct((B, D), jnp.float32),
               jax.ShapeDtypeStruct((B, D), jnp.float32)],
)


def kernel(emb_2d, emb_3d, segment_ids):
    seg = segment_ids.astype(jnp.int32)
    i = jnp.arange(N, dtype=jnp.int32)
    boundary = jnp.concatenate(
        [jnp.ones((1,), jnp.bool_), seg[1:] != seg[:-1]])
    run_start = lax.cummax(jnp.where(boundary, i, 0))
    pos = i - run_start
    tgt = jnp.where(pos < MAX_NODES, seg, TRASH).reshape(NW, NCHUNK, CHUNK)
    p2, p3, cnt = _sc_pool(emb_2d, emb_3d, tgt)
    out2, out3 = _combine(p2, p3, cnt)
    return (out2, out3)
